# baseline (speedup 1.0000x reference)
import functools

import jax
import jax.numpy as jnp
from jax import lax
from jax.experimental import pallas as pl
from jax.experimental.pallas import tpu as pltpu
from jax.experimental.pallas import tpu_sc as plsc

NUM_TYPES = 1000000
CHANNELS = 64
B = 16384
L = 200

NC = 2
NS = 16
NW = NC * NS

GW = 128
N_FLAT = B * L
K = 5
KI = K * GW
PER_W = N_FLAT // NW
NCH = PER_W // KI


def _sc_gather(types_flat, table):
    mesh = plsc.VectorSubcoreMesh(core_axis_name="c", subcore_axis_name="s")

    @functools.partial(
        pl.kernel,
        mesh=mesh,
        out_type=jax.ShapeDtypeStruct((N_FLAT, 2 * CHANNELS), jnp.float32),
        scratch_types=[
            pltpu.VMEM((2, KI), jnp.int32),
            pltpu.VMEM((2, KI, CHANNELS), jnp.float32),
            pltpu.SemaphoreType.DMA,
            pltpu.SemaphoreType.DMA,
            pltpu.SemaphoreType.DMA,
            pltpu.SemaphoreType.DMA,
            pltpu.SemaphoreType.DMA,
        ],
        compiler_params=pltpu.CompilerParams(use_tc_tiling_on_sc=False),
    )
    def run(idx_hbm, table_hbm, out_hbm, idx_v, rows_v,
            sem_g0, sem_g1, sem_o0, sem_o1, sem_i):
        wid = lax.axis_index("s") * NC + lax.axis_index("c")
        w_base = wid * PER_W
        sems_g = (sem_g0, sem_g1)
        sems_o = (sem_o0, sem_o1)

        def prefetch_idx(c, buf):
            base = w_base + c * KI
            pltpu.async_copy(idx_hbm.at[pl.ds(base, KI)], idx_v.at[buf], sem_i)

        def wait_idx(c, buf):
            base = w_base + c * KI
            pltpu.make_async_copy(
                idx_hbm.at[pl.ds(base, KI)], idx_v.at[buf], sem_i
            ).wait()

        def fire_gathers(c, buf):
            for j in range(K):
                pltpu.async_copy(
                    table_hbm.at[idx_v.at[buf, pl.ds(j * GW, GW)]],
                    rows_v.at[buf, pl.ds(j * GW, GW)],
                    sems_g[buf],
                )

        def drain_gathers(c, buf):
            for j in range(K):
                pltpu.make_async_copy(
                    table_hbm.at[idx_v.at[buf, pl.ds(j * GW, GW)]],
                    rows_v.at[buf, pl.ds(j * GW, GW)],
                    sems_g[buf],
                ).wait()

        def out_copy(c, buf):
            base = w_base + c * KI
            pltpu.async_copy(
                rows_v.at[buf],
                out_hbm.at[pl.ds(base, KI), pl.ds(0, CHANNELS)],
                sems_o[buf],
            )

        def wait_out(c, buf):
            base = w_base + c * KI
            pltpu.make_async_copy(
                rows_v.at[buf],
                out_hbm.at[pl.ds(base, KI), pl.ds(0, CHANNELS)],
                sems_o[buf],
            ).wait()

        prefetch_idx(0, 0)
        prefetch_idx(1, 1)
        wait_idx(0, 0)
        fire_gathers(0, 0)

        def step(c, carry):
            def body_for(buf):
                nbuf = 1 - buf

                @pl.when(c + 1 < NCH)
                def _():
                    @pl.when(c >= 1)
                    def _():
                        wait_out(c - 1, nbuf)

                    wait_idx(c + 1, nbuf)
                    fire_gathers(c + 1, nbuf)

                drain_gathers(c, buf)
                out_copy(c, buf)

                @pl.when(c + 2 < NCH)
                def _():
                    prefetch_idx(c + 2, buf)

            lax.cond(c % 2 == 0, lambda: body_for(0), lambda: body_for(1))
            return carry

        lax.fori_loop(0, NCH, step, 0)
        wait_out(NCH - 2, (NCH - 2) % 2)
        wait_out(NCH - 1, (NCH - 1) % 2)

    return run(types_flat, table)


def kernel(types, table):
    out = _sc_gather(types.reshape(N_FLAT), table)
    return out[:, :CHANNELS].reshape(B, L, CHANNELS)
